# baseline (speedup 1.0000x reference)
import functools

import jax
import jax.numpy as jnp
from jax import lax
from jax.experimental import pallas as pl
from jax.experimental.pallas import tpu as pltpu
from jax.experimental.pallas import tpu_sc as plsc

DIM = 2048
NUM_EXPERTS = 64
NUM_PATTERNS = 16
TOP_K = 2
BATCH = 4
SEQ = 2048
TOKENS = BATCH * SEQ

TS = 1024

NC, NS, LANES = 2, 16, 16
NW = NC * NS
TPW = TOKENS // NW
WPB = SEQ // TPW
NSEG = NUM_EXPERTS // LANES


def _matmul_kernel(x_ref, w_ref, out_ref):
    out_ref[...] = lax.dot_general(
        x_ref[...], w_ref[...],
        dimension_numbers=(((1,), (1,)), ((), ())),
        preferred_element_type=jnp.float32,
    )


def _sc_route(cl_hbm, pw_hbm, ids_hbm, rl_out, wts_out, idx_out,
              cl_v, pw_v, ids_v, wts_v, idx_v):
    wid = lax.axis_index("s") * NC + lax.axis_index("c")
    base = wid * TPW
    pltpu.sync_copy(cl_hbm.at[pl.ds(base, TPW)], cl_v)
    pltpu.sync_copy(pw_hbm, pw_v)
    pltpu.sync_copy(ids_hbm, ids_v)

    b = wid // WPB
    iota = lax.iota(jnp.int32, LANES)
    ids_vec = ids_v[...]
    pid_s = jnp.max(jnp.where(iota == b, ids_vec, 0), axis=0)
    pid = jnp.full((LANES,), pid_s, jnp.int32)
    bias_seg = [plsc.load_gather(pw_v, [pid, iota + LANES * j])
                for j in range(NSEG)]
    neg = jnp.float32(-jnp.inf)
    zeros = jnp.zeros((LANES,), jnp.int32)
    ones = jnp.ones((LANES,), jnp.int32)

    def body(g, carry):
        tv = g * LANES + iota
        best = jnp.full((LANES,), neg)
        sec = jnp.full((LANES,), neg)
        bidx = zeros
        sidx = zeros
        for e in range(NUM_EXPERTS):
            ev = jnp.full((LANES,), e, jnp.int32)
            bias_e = jnp.max(jnp.where(iota == (e % LANES),
                                       bias_seg[e // LANES], neg), axis=0)
            v = plsc.load_gather(cl_v, [tv, ev]) + bias_e
            plsc.store_scatter(cl_v, [tv, ev], v)
            isnew = v > best
            gtsec = v > sec
            sec, sidx = (jnp.where(isnew, best, jnp.where(gtsec, v, sec)),
                         jnp.where(isnew, bidx, jnp.where(gtsec, ev, sidx)))
            best = jnp.where(isnew, v, best)
            bidx = jnp.where(isnew, ev, bidx)
        expv = jnp.exp(sec - best)
        w1 = 1.0 / (1.0 + expv)
        plsc.store_scatter(wts_v, [tv, zeros], w1)
        plsc.store_scatter(wts_v, [tv, ones], 1.0 - w1)
        plsc.store_scatter(idx_v, [tv, zeros], bidx)
        plsc.store_scatter(idx_v, [tv, ones], sidx)
        return carry

    lax.fori_loop(0, TPW // LANES, body, 0)
    pltpu.sync_copy(cl_v, rl_out.at[pl.ds(base, TPW)])
    pltpu.sync_copy(wts_v, wts_out.at[pl.ds(base, TPW)])
    pltpu.sync_copy(idx_v, idx_out.at[pl.ds(base, TPW)])


_sc_route_call = functools.partial(
    pl.kernel,
    out_type=(
        jax.ShapeDtypeStruct((TOKENS, NUM_EXPERTS), jnp.float32),
        jax.ShapeDtypeStruct((TOKENS, TOP_K), jnp.float32),
        jax.ShapeDtypeStruct((TOKENS, TOP_K), jnp.int32),
    ),
    mesh=plsc.VectorSubcoreMesh(core_axis_name="c", subcore_axis_name="s"),
    compiler_params=pltpu.CompilerParams(needs_layout_passes=False),
    scratch_types=[
        pltpu.VMEM((TPW, NUM_EXPERTS), jnp.float32),
        pltpu.VMEM((NUM_PATTERNS, NUM_EXPERTS), jnp.float32),
        pltpu.VMEM((LANES,), jnp.int32),
        pltpu.VMEM((TPW, TOP_K), jnp.float32),
        pltpu.VMEM((TPW, TOP_K), jnp.int32),
    ],
)(_sc_route)


@jax.jit
def kernel(x, pattern_ids, content_w, pattern_w):
    x2d = x.reshape(TOKENS, DIM)
    content_logits = pl.pallas_call(
        _matmul_kernel,
        grid=(TOKENS // TS,),
        in_specs=[
            pl.BlockSpec((TS, DIM), lambda s: (s, 0)),
            pl.BlockSpec((NUM_EXPERTS, DIM), lambda s: (0, 0)),
        ],
        out_specs=pl.BlockSpec((TS, NUM_EXPERTS), lambda s: (s, 0)),
        out_shape=jax.ShapeDtypeStruct((TOKENS, NUM_EXPERTS), jnp.float32),
    )(x2d, content_w)

    ids16 = jnp.zeros((LANES,), jnp.int32).at[:BATCH].set(
        pattern_ids.astype(jnp.int32))
    rl, wts, idx = _sc_route_call(content_logits, pattern_w, ids16)
    return (wts.reshape(BATCH, SEQ, TOP_K),
            idx.reshape(BATCH, SEQ, TOP_K),
            rl.reshape(BATCH, SEQ, NUM_EXPERTS))


# speedup vs baseline: 1.3347x; 1.3347x over previous
import functools

import jax
import jax.numpy as jnp
from jax import lax
from jax.experimental import pallas as pl
from jax.experimental.pallas import tpu as pltpu
from jax.experimental.pallas import tpu_sc as plsc

DIM = 2048
NUM_EXPERTS = 64
NUM_PATTERNS = 16
TOP_K = 2
BATCH = 4
SEQ = 2048
TOKENS = BATCH * SEQ

TS = 1024

NC, NS, LANES = 2, 16, 16
NSEG = NUM_EXPERTS // LANES


def _sc_bias_gather(pw_hbm, ids_hbm, bias_out, pw_v, ids_v, bias_v):
    wid = lax.axis_index("s") * NC + lax.axis_index("c")

    @pl.when(wid == 0)
    def _():
        pltpu.sync_copy(pw_hbm, pw_v)
        pltpu.sync_copy(ids_hbm, ids_v)
        iota = lax.iota(jnp.int32, LANES)
        ids_vec = ids_v[...]
        for b in range(BATCH):
            pid_s = jnp.max(jnp.where(iota == b, ids_vec, 0), axis=0)
            pid = jnp.full((LANES,), pid_s, jnp.int32)
            for j in range(NSEG):
                bias_v[b, LANES * j:LANES * (j + 1)] = plsc.load_gather(
                    pw_v, [pid, iota + LANES * j])
        pltpu.sync_copy(bias_v, bias_out)


_sc_bias_call = functools.partial(
    pl.kernel,
    out_type=jax.ShapeDtypeStruct((BATCH, NUM_EXPERTS), jnp.float32),
    mesh=plsc.VectorSubcoreMesh(core_axis_name="c", subcore_axis_name="s"),
    compiler_params=pltpu.CompilerParams(needs_layout_passes=False),
    scratch_types=[
        pltpu.VMEM((NUM_PATTERNS, NUM_EXPERTS), jnp.float32),
        pltpu.VMEM((LANES,), jnp.int32),
        pltpu.VMEM((BATCH, NUM_EXPERTS), jnp.float32),
    ],
)(_sc_bias_gather)


def _router_kernel(x_ref, w_ref, bias_ref, logits_ref, wts_ref, idx_ref):
    xt = x_ref[0]
    logits = lax.dot_general(
        xt, w_ref[...],
        dimension_numbers=(((1,), (1,)), ((), ())),
        preferred_element_type=jnp.float32,
    )
    logits = logits + bias_ref[0]
    logits_ref[0] = logits

    eids = lax.broadcasted_iota(jnp.int32, (TS, NUM_EXPERTS), 1)
    m1 = jnp.max(logits, axis=1)
    i1 = jnp.argmax(logits, axis=1).astype(jnp.int32)
    masked = jnp.where(eids == i1[:, None], -jnp.inf, logits)
    m2 = jnp.max(masked, axis=1)
    i2 = jnp.argmax(masked, axis=1).astype(jnp.int32)
    e = jnp.exp(m2 - m1)
    w1 = 1.0 / (1.0 + e)
    w2 = e / (1.0 + e)
    wts_ref[0] = jnp.stack([w1, w2], axis=-1)
    idx_ref[0] = jnp.stack([i1, i2], axis=-1)


@jax.jit
def kernel(x, pattern_ids, content_w, pattern_w):
    ids16 = jnp.zeros((LANES,), jnp.int32).at[:BATCH].set(
        pattern_ids.astype(jnp.int32))
    bias = _sc_bias_call(pattern_w, ids16)
    bias = bias.reshape(BATCH, 1, NUM_EXPERTS)

    grid = (BATCH, SEQ // TS)
    out_shapes = (
        jax.ShapeDtypeStruct((BATCH, SEQ, NUM_EXPERTS), jnp.float32),
        jax.ShapeDtypeStruct((BATCH, SEQ, TOP_K), jnp.float32),
        jax.ShapeDtypeStruct((BATCH, SEQ, TOP_K), jnp.int32),
    )
    logits, wts, idx = pl.pallas_call(
        _router_kernel,
        grid=grid,
        in_specs=[
            pl.BlockSpec((1, TS, DIM), lambda b, s: (b, s, 0)),
            pl.BlockSpec((NUM_EXPERTS, DIM), lambda b, s: (0, 0)),
            pl.BlockSpec((1, 1, NUM_EXPERTS), lambda b, s: (b, 0, 0)),
        ],
        out_specs=(
            pl.BlockSpec((1, TS, NUM_EXPERTS), lambda b, s: (b, s, 0)),
            pl.BlockSpec((1, TS, TOP_K), lambda b, s: (b, s, 0)),
            pl.BlockSpec((1, TS, TOP_K), lambda b, s: (b, s, 0)),
        ),
        out_shape=out_shapes,
    )(x, content_w, bias)
    return (wts, idx, logits)


# speedup vs baseline: 2.1498x; 1.6107x over previous
import functools

import jax
import jax.numpy as jnp
from jax.experimental import pallas as pl
from jax.experimental.pallas import tpu as pltpu

DIM = 2048
NUM_EXPERTS = 64
NUM_PATTERNS = 16
TOP_K = 2
BATCH = 4
SEQ = 2048

TS = 1024


def _router_kernel(pattern_ids_ref, x_ref, w_ref, pattern_w_ref,
                   logits_ref, wts_ref, idx_ref):
    b = pl.program_id(0)
    xt = x_ref[0]
    logits = jax.lax.dot_general(
        xt, w_ref[...],
        dimension_numbers=(((1,), (1,)), ((), ())),
        preferred_element_type=jnp.float32,
    )
    pid = pattern_ids_ref[b]
    bias = pattern_w_ref[pid, :]
    logits = logits + bias[None, :]
    logits_ref[0] = logits

    eids = jax.lax.broadcasted_iota(jnp.int32, (TS, NUM_EXPERTS), 1)
    m1 = jnp.max(logits, axis=1)
    i1 = jnp.argmax(logits, axis=1).astype(jnp.int32)
    masked = jnp.where(eids == i1[:, None], -jnp.inf, logits)
    m2 = jnp.max(masked, axis=1)
    i2 = jnp.argmax(masked, axis=1).astype(jnp.int32)
    e = jnp.exp(m2 - m1)
    w1 = 1.0 / (1.0 + e)
    w2 = e / (1.0 + e)
    wts_ref[0] = jnp.stack([w1, w2], axis=-1)
    idx_ref[0] = jnp.stack([i1, i2], axis=-1)


@jax.jit
def kernel(x, pattern_ids, content_w, pattern_w):
    grid = (BATCH, SEQ // TS)
    out_shapes = (
        jax.ShapeDtypeStruct((BATCH, SEQ, NUM_EXPERTS), jnp.float32),
        jax.ShapeDtypeStruct((BATCH, SEQ, TOP_K), jnp.float32),
        jax.ShapeDtypeStruct((BATCH, SEQ, TOP_K), jnp.int32),
    )
    logits, wts, idx = pl.pallas_call(
        _router_kernel,
        grid=grid,
        in_specs=[
            pl.BlockSpec(memory_space=pltpu.SMEM),
            pl.BlockSpec((1, TS, DIM), lambda b, s: (b, s, 0)),
            pl.BlockSpec((NUM_EXPERTS, DIM), lambda b, s: (0, 0)),
            pl.BlockSpec((NUM_PATTERNS, NUM_EXPERTS), lambda b, s: (0, 0)),
        ],
        out_specs=(
            pl.BlockSpec((1, TS, NUM_EXPERTS), lambda b, s: (b, s, 0)),
            pl.BlockSpec((1, TS, TOP_K), lambda b, s: (b, s, 0)),
            pl.BlockSpec((1, TS, TOP_K), lambda b, s: (b, s, 0)),
        ),
        out_shape=out_shapes,
    )(pattern_ids.astype(jnp.int32), x, content_w, pattern_w)
    return (wts, idx, logits)


# speedup vs baseline: 2.1665x; 1.0077x over previous
import functools

import jax
import jax.numpy as jnp
from jax.experimental import pallas as pl
from jax.experimental.pallas import tpu as pltpu

DIM = 2048
NUM_EXPERTS = 64
NUM_PATTERNS = 16
TOP_K = 2
BATCH = 4
SEQ = 2048

TS = 1024


def _router_kernel(pattern_ids_ref, x_ref, w_ref, pattern_w_ref,
                   logits_ref, wts_ref, idx_ref):
    b = pl.program_id(0)
    xt = x_ref[0]
    logits = jax.lax.dot_general(
        xt, w_ref[...],
        dimension_numbers=(((1,), (1,)), ((), ())),
        preferred_element_type=jnp.float32,
    )
    pid = pattern_ids_ref[b]
    bias = pattern_w_ref[pid, :]
    logits = logits + bias[None, :]
    logits_ref[0] = logits

    eids = jax.lax.broadcasted_iota(jnp.int32, (TS, NUM_EXPERTS), 1)
    m1 = jnp.max(logits, axis=1)
    i1 = jnp.argmax(logits, axis=1).astype(jnp.int32)
    masked = jnp.where(eids == i1[:, None], -jnp.inf, logits)
    m2 = jnp.max(masked, axis=1)
    i2 = jnp.argmax(masked, axis=1).astype(jnp.int32)
    e = jnp.exp(m2 - m1)
    w1 = 1.0 / (1.0 + e)
    w2 = 1.0 - w1
    wts_ref[0] = jnp.stack([w1, w2], axis=-1)
    idx_ref[0] = jnp.stack([i1, i2], axis=-1)


@jax.jit
def kernel(x, pattern_ids, content_w, pattern_w):
    grid = (BATCH, SEQ // TS)
    out_shapes = (
        jax.ShapeDtypeStruct((BATCH, SEQ, NUM_EXPERTS), jnp.float32),
        jax.ShapeDtypeStruct((BATCH, SEQ, TOP_K), jnp.float32),
        jax.ShapeDtypeStruct((BATCH, SEQ, TOP_K), jnp.int32),
    )
    logits, wts, idx = pl.pallas_call(
        _router_kernel,
        grid=grid,
        in_specs=[
            pl.BlockSpec(memory_space=pltpu.SMEM),
            pl.BlockSpec((1, TS, DIM), lambda b, s: (b, s, 0)),
            pl.BlockSpec((NUM_EXPERTS, DIM), lambda b, s: (0, 0)),
            pl.BlockSpec((NUM_PATTERNS, NUM_EXPERTS), lambda b, s: (0, 0)),
        ],
        out_specs=(
            pl.BlockSpec((1, TS, NUM_EXPERTS), lambda b, s: (b, s, 0)),
            pl.BlockSpec((1, TS, TOP_K), lambda b, s: (b, s, 0)),
            pl.BlockSpec((1, TS, TOP_K), lambda b, s: (b, s, 0)),
        ),
        out_shape=out_shapes,
    )(pattern_ids.astype(jnp.int32), x, content_w, pattern_w)
    return (wts, idx, logits)


# speedup vs baseline: 2.1675x; 1.0005x over previous
import jax
import jax.numpy as jnp
from jax.experimental import pallas as pl
from jax.experimental.pallas import tpu as pltpu

DIM = 2048
NUM_EXPERTS = 64
NUM_PATTERNS = 16
TOP_K = 2
BATCH = 4
SEQ = 2048

TS = 1024


def _router_kernel(pattern_ids_ref, x_ref, w_ref, pattern_w_ref,
                   logits_ref, wts_ref, idx_ref):
    b = pl.program_id(0)
    xt = x_ref[0]
    logits = jax.lax.dot_general(
        xt, w_ref[...],
        dimension_numbers=(((1,), (1,)), ((), ())),
        preferred_element_type=jnp.float32,
    )
    pid = pattern_ids_ref[b]
    bias = pattern_w_ref[pid, :]
    logits = logits + bias[None, :]
    logits_ref[0] = logits

    eids = jax.lax.broadcasted_iota(jnp.int32, (TS, NUM_EXPERTS), 1)
    m1 = jnp.max(logits, axis=1)
    i1 = jnp.argmax(logits, axis=1).astype(jnp.int32)
    masked = jnp.where(eids == i1[:, None], -jnp.inf, logits)
    m2 = jnp.max(masked, axis=1)
    i2 = jnp.argmax(masked, axis=1).astype(jnp.int32)
    e = jnp.exp(m2 - m1)
    w1 = 1.0 / (1.0 + e)
    w2 = 1.0 - w1
    wts_ref[0] = jnp.stack([w1, w2], axis=-1)
    idx_ref[0] = jnp.stack([i1, i2], axis=-1)


@jax.jit
def kernel(x, pattern_ids, content_w, pattern_w):
    grid = (BATCH, SEQ // TS)
    out_shapes = (
        jax.ShapeDtypeStruct((BATCH, SEQ, NUM_EXPERTS), jnp.float32),
        jax.ShapeDtypeStruct((BATCH, SEQ, TOP_K), jnp.float32),
        jax.ShapeDtypeStruct((BATCH, SEQ, TOP_K), jnp.int32),
    )
    logits, wts, idx = pl.pallas_call(
        _router_kernel,
        grid=grid,
        in_specs=[
            pl.BlockSpec(memory_space=pltpu.SMEM),
            pl.BlockSpec((1, TS, DIM), lambda b, s: (b, s, 0)),
            pl.BlockSpec((NUM_EXPERTS, DIM), lambda b, s: (0, 0)),
            pl.BlockSpec((NUM_PATTERNS, NUM_EXPERTS), lambda b, s: (0, 0)),
        ],
        out_specs=(
            pl.BlockSpec((1, TS, NUM_EXPERTS), lambda b, s: (b, s, 0)),
            pl.BlockSpec((1, TS, TOP_K), lambda b, s: (b, s, 0)),
            pl.BlockSpec((1, TS, TOP_K), lambda b, s: (b, s, 0)),
        ),
        out_shape=out_shapes,
    )(pattern_ids.astype(jnp.int32), x, content_w, pattern_w)
    return (wts, idx, logits)
